# baseline (speedup 1.0000x reference)
import functools

import jax
import jax.numpy as jnp
from jax import lax
from jax.experimental import pallas as pl
from jax.experimental.pallas import tpu as pltpu
from jax.experimental.pallas import tpu_sc as plsc

BATCH = 16384
EMBED_DIM = 32
SUPER = 128
CHUNK = 128


def _relayout_body(tT_ref, out_ref):
    x = tT_ref[...]
    c = x.shape[1]
    eye = jnp.float32(
        lax.broadcasted_iota(jnp.int32, (EMBED_DIM, EMBED_DIM), 0)
        == lax.broadcasted_iota(jnp.int32, (EMBED_DIM, EMBED_DIM), 1))
    t = lax.dot_general(x, eye, (((0,), (0,)), ((), ())),
                        preferred_element_type=jnp.float32,
                        precision=lax.Precision.HIGHEST)
    t = t.reshape(c // 4, 4, EMBED_DIM)
    out_ref[...] = jnp.concatenate([t[:, k, :] for k in range(4)], axis=-1)


def _tc_relayout(tabT, block_c):
    rows = tabT.shape[1]
    n_blocks = pl.cdiv(rows, block_c)
    return pl.pallas_call(
        _relayout_body,
        grid=(n_blocks,),
        in_specs=[pl.BlockSpec((EMBED_DIM, block_c), lambda i: (0, i))],
        out_specs=pl.BlockSpec((block_c // 4, SUPER), lambda i: (i, 0)),
        out_shape=jax.ShapeDtypeStruct((rows // 4, SUPER), jnp.float32),
    )(tabT)


def _sc_gather(item_id3, org_id3, item_tab4, org_tab4, n_workers, n_chunks):
    b_per_w = n_chunks * CHUNK
    mesh = plsc.VectorSubcoreMesh(core_axis_name="c", subcore_axis_name="s")

    @functools.partial(
        pl.kernel,
        out_type=(
            jax.ShapeDtypeStruct((BATCH, SUPER), jnp.float32),
            jax.ShapeDtypeStruct((BATCH, SUPER), jnp.float32),
        ),
        mesh=mesh,
        scratch_types=[
            pltpu.VMEM((n_chunks, CHUNK), jnp.int32),
            pltpu.VMEM((n_chunks, CHUNK), jnp.int32),
            pltpu.VMEM((2, CHUNK, SUPER), jnp.float32),
            pltpu.VMEM((2, CHUNK, SUPER), jnp.float32),
            pltpu.SemaphoreType.DMA,
            pltpu.SemaphoreType.DMA,
        ],
    )
    def k(iid_hbm, oid_hbm, itab_hbm, otab_hbm, iout_hbm, oout_hbm,
          iidx_v, oidx_v, ibuf_v, obuf_v, gsem, osem):
        wid = lax.axis_index("s") * 2 + lax.axis_index("c")
        base = wid * b_per_w
        pltpu.sync_copy(iid_hbm.at[wid], iidx_v)
        pltpu.sync_copy(oid_hbm.at[wid], oidx_v)
        outs = []
        for j in range(n_chunks):
            s = j % 2
            if j >= 2:
                outs[2 * (j - 2)].wait()
                outs[2 * (j - 2) + 1].wait()
            g1 = pltpu.async_copy(
                itab_hbm.at[iidx_v.at[j]], ibuf_v.at[s], gsem)
            g2 = pltpu.async_copy(
                otab_hbm.at[oidx_v.at[j]], obuf_v.at[s], gsem)
            g1.wait()
            g2.wait()
            dst = pl.ds(base + j * CHUNK, CHUNK)
            outs.append(pltpu.async_copy(
                ibuf_v.at[s], iout_hbm.at[dst], osem))
            outs.append(pltpu.async_copy(
                obuf_v.at[s], oout_hbm.at[dst], osem))
        for c in outs[-4:]:
            c.wait()

    return k(item_id3, org_id3, item_tab4, org_tab4)


def _select32(x128, sel):
    out = None
    for kk in range(4):
        part = jnp.where(sel == kk, x128[:, kk * 32:(kk + 1) * 32], 0.0)
        out = part if out is None else out + part
    return out


def _mlp_body(ig_ref, og_ref, isel_ref, osel_ref,
              w1_ref, b1_ref, w2_ref, b2_ref, w3_ref, b3_ref, out_ref):
    iv = _select32(ig_ref[...], isel_ref[...])
    ov = _select32(og_ref[...], osel_ref[...])
    c = jnp.concatenate([iv, ov], axis=-1)
    x = jnp.maximum(
        jnp.dot(c, w1_ref[...], preferred_element_type=jnp.float32)
        + b1_ref[...], 0.0)
    x = jnp.maximum(
        jnp.dot(x, w2_ref[...], preferred_element_type=jnp.float32)
        + b2_ref[...], 0.0)
    y = jnp.dot(x, w3_ref[...], preferred_element_type=jnp.float32) + b3_ref[...]
    out_ref[...] = jax.nn.sigmoid(y)


def _tc_mlp(ig, og, isel, osel, W1, b1, W2, b2, W3, b3, block_b=2048):
    n_blocks = BATCH // block_b
    full = lambda shape: pl.BlockSpec(shape, lambda i: (0, 0))
    return pl.pallas_call(
        _mlp_body,
        grid=(n_blocks,),
        in_specs=[
            pl.BlockSpec((block_b, SUPER), lambda i: (i, 0)),
            pl.BlockSpec((block_b, SUPER), lambda i: (i, 0)),
            pl.BlockSpec((block_b, 1), lambda i: (i, 0)),
            pl.BlockSpec((block_b, 1), lambda i: (i, 0)),
            full((2 * EMBED_DIM, 128)),
            full((1, 128)),
            full((128, 64)),
            full((1, 64)),
            full((64, 1)),
            full((1, 1)),
        ],
        out_specs=pl.BlockSpec((block_b, 1), lambda i: (i, 0)),
        out_shape=jax.ShapeDtypeStruct((BATCH, 1), jnp.float32),
    )(ig, og, isel, osel, W1, b1.reshape(1, -1), W2, b2.reshape(1, -1), W3,
      b3.reshape(1, -1))


def kernel(item_id, org_id, item_table, org_table, W1, b1, W2, b2, W3, b3):
    info = plsc.get_sparse_core_info()
    n_workers = info.num_cores * info.num_subcores
    n_chunks = BATCH // (n_workers * CHUNK)
    item_id = item_id.astype(jnp.int32)
    org_id = org_id.astype(jnp.int32)
    item_id3 = (item_id >> 2).reshape(n_workers, n_chunks, CHUNK)
    org_id3 = (org_id >> 2).reshape(n_workers, n_chunks, CHUNK)
    item_tab4 = _tc_relayout(item_table.T, 16384)
    org_tab4 = _tc_relayout(org_table.T, 16384)
    ig, og = _sc_gather(item_id3, org_id3, item_tab4, org_tab4,
                        n_workers, n_chunks)
    isel = (item_id & 3).reshape(BATCH, 1)
    osel = (org_id & 3).reshape(BATCH, 1)
    return _tc_mlp(ig, og, isel, osel, W1, b1, W2, b2, W3, b3)


# speedup vs baseline: 4.0230x; 4.0230x over previous
import functools

import jax
import jax.numpy as jnp
from jax import lax
from jax.experimental import pallas as pl
from jax.experimental.pallas import tpu as pltpu
from jax.experimental.pallas import tpu_sc as plsc

BATCH = 16384
EMBED_DIM = 32
SUPER = 128
CHUNK = 128


BLK = 16384
QTR = BLK // 4


def _relayout_body(rows, tT_ref, out_ref):
    x = tT_ref[...]
    valid = rows - pl.program_id(0) * BLK
    col = lax.broadcasted_iota(jnp.int32, x.shape, 1)
    x = jnp.where(col < valid, x, 0.0)
    xs = jnp.concatenate([x[:, q * QTR:(q + 1) * QTR] for q in range(4)],
                         axis=0)
    eye = jnp.float32(
        lax.broadcasted_iota(jnp.int32, (SUPER, SUPER), 0)
        == lax.broadcasted_iota(jnp.int32, (SUPER, SUPER), 1))
    hi = xs.astype(jnp.bfloat16)
    lo = (xs - hi.astype(jnp.float32)).astype(jnp.bfloat16)
    dims = (((0,), (0,)), ((), ()))
    out_ref[...] = (
        lax.dot_general(hi, eye.astype(jnp.bfloat16), dims,
                        preferred_element_type=jnp.float32)
        + lax.dot_general(lo, eye.astype(jnp.bfloat16), dims,
                          preferred_element_type=jnp.float32))


def _tc_relayout(tabT):
    rows = tabT.shape[1]
    n_blocks = pl.cdiv(rows, BLK)
    return pl.pallas_call(
        functools.partial(_relayout_body, rows),
        grid=(n_blocks,),
        in_specs=[pl.BlockSpec((EMBED_DIM, BLK), lambda i: (0, i))],
        out_specs=pl.BlockSpec((QTR, SUPER), lambda i: (i, 0)),
        out_shape=jax.ShapeDtypeStruct((n_blocks * QTR, SUPER), jnp.float32),
    )(tabT)


def _sc_gather(item_id3, org_id3, item_tab4, org_tab4, n_workers, n_chunks):
    b_per_w = n_chunks * CHUNK
    mesh = plsc.VectorSubcoreMesh(core_axis_name="c", subcore_axis_name="s")

    @functools.partial(
        pl.kernel,
        out_type=(
            jax.ShapeDtypeStruct((BATCH, SUPER), jnp.float32),
            jax.ShapeDtypeStruct((BATCH, SUPER), jnp.float32),
        ),
        mesh=mesh,
        scratch_types=[
            pltpu.VMEM((n_chunks, CHUNK), jnp.int32),
            pltpu.VMEM((n_chunks, CHUNK), jnp.int32),
            pltpu.VMEM((2, CHUNK, SUPER), jnp.float32),
            pltpu.VMEM((2, CHUNK, SUPER), jnp.float32),
            pltpu.SemaphoreType.DMA,
            pltpu.SemaphoreType.DMA,
        ],
    )
    def k(iid_hbm, oid_hbm, itab_hbm, otab_hbm, iout_hbm, oout_hbm,
          iidx_v, oidx_v, ibuf_v, obuf_v, gsem, osem):
        wid = lax.axis_index("s") * 2 + lax.axis_index("c")
        base = wid * b_per_w
        pltpu.sync_copy(iid_hbm.at[wid], iidx_v)
        pltpu.sync_copy(oid_hbm.at[wid], oidx_v)
        outs = []
        for j in range(n_chunks):
            s = j % 2
            if j >= 2:
                outs[2 * (j - 2)].wait()
                outs[2 * (j - 2) + 1].wait()
            g1 = pltpu.async_copy(
                itab_hbm.at[iidx_v.at[j]], ibuf_v.at[s], gsem)
            g2 = pltpu.async_copy(
                otab_hbm.at[oidx_v.at[j]], obuf_v.at[s], gsem)
            g1.wait()
            g2.wait()
            dst = pl.ds(base + j * CHUNK, CHUNK)
            outs.append(pltpu.async_copy(
                ibuf_v.at[s], iout_hbm.at[dst], osem))
            outs.append(pltpu.async_copy(
                obuf_v.at[s], oout_hbm.at[dst], osem))
        for c in outs[-4:]:
            c.wait()

    return k(item_id3, org_id3, item_tab4, org_tab4)


def _select32(x128, sel):
    out = None
    for kk in range(4):
        part = jnp.where(sel == kk, x128[:, kk * 32:(kk + 1) * 32], 0.0)
        out = part if out is None else out + part
    return out


def _mlp_body(ig_ref, og_ref, isel_ref, osel_ref,
              w1_ref, b1_ref, w2_ref, b2_ref, w3_ref, b3_ref, out_ref):
    iv = _select32(ig_ref[...], isel_ref[...])
    ov = _select32(og_ref[...], osel_ref[...])
    c = jnp.concatenate([iv, ov], axis=-1)
    x = jnp.maximum(
        jnp.dot(c, w1_ref[...], preferred_element_type=jnp.float32)
        + b1_ref[...], 0.0)
    x = jnp.maximum(
        jnp.dot(x, w2_ref[...], preferred_element_type=jnp.float32)
        + b2_ref[...], 0.0)
    y = jnp.dot(x, w3_ref[...], preferred_element_type=jnp.float32) + b3_ref[...]
    out_ref[...] = jax.nn.sigmoid(y)


def _tc_mlp(ig, og, isel, osel, W1, b1, W2, b2, W3, b3, block_b=2048):
    n_blocks = BATCH // block_b
    full = lambda shape: pl.BlockSpec(shape, lambda i: (0, 0))
    return pl.pallas_call(
        _mlp_body,
        grid=(n_blocks,),
        in_specs=[
            pl.BlockSpec((block_b, SUPER), lambda i: (i, 0)),
            pl.BlockSpec((block_b, SUPER), lambda i: (i, 0)),
            pl.BlockSpec((block_b, 1), lambda i: (i, 0)),
            pl.BlockSpec((block_b, 1), lambda i: (i, 0)),
            full((2 * EMBED_DIM, 128)),
            full((1, 128)),
            full((128, 64)),
            full((1, 64)),
            full((64, 1)),
            full((1, 1)),
        ],
        out_specs=pl.BlockSpec((block_b, 1), lambda i: (i, 0)),
        out_shape=jax.ShapeDtypeStruct((BATCH, 1), jnp.float32),
    )(ig, og, isel, osel, W1, b1.reshape(1, -1), W2, b2.reshape(1, -1), W3,
      b3.reshape(1, -1))


def kernel(item_id, org_id, item_table, org_table, W1, b1, W2, b2, W3, b3):
    info = plsc.get_sparse_core_info()
    n_workers = info.num_cores * info.num_subcores
    n_chunks = BATCH // (n_workers * CHUNK)
    item_id = item_id.astype(jnp.int32)
    org_id = org_id.astype(jnp.int32)
    isup = ((item_id // BLK) * QTR) + (item_id % QTR)
    osup = ((org_id // BLK) * QTR) + (org_id % QTR)
    item_id3 = isup.reshape(n_workers, n_chunks, CHUNK)
    org_id3 = osup.reshape(n_workers, n_chunks, CHUNK)
    item_tab4 = _tc_relayout(item_table.T)
    org_tab4 = _tc_relayout(org_table.T)
    ig, og = _sc_gather(item_id3, org_id3, item_tab4, org_tab4,
                        n_workers, n_chunks)
    isel = ((item_id % BLK) // QTR).reshape(BATCH, 1)
    osel = ((org_id % BLK) // QTR).reshape(BATCH, 1)
    return _tc_mlp(ig, og, isel, osel, W1, b1, W2, b2, W3, b3)


# speedup vs baseline: 4.5047x; 1.1197x over previous
import functools

import jax
import jax.numpy as jnp
from jax import lax
from jax.experimental import pallas as pl
from jax.experimental.pallas import tpu as pltpu
from jax.experimental.pallas import tpu_sc as plsc

BATCH = 16384
EMBED_DIM = 32
SUPER = 128
CHUNK = 128


BLK = 16384
QTR = BLK // 4


def _relayout_body(rows, tT_ref, out_ref):
    x = tT_ref[...]
    valid = rows - pl.program_id(0) * BLK
    col = lax.broadcasted_iota(jnp.int32, x.shape, 1)
    x = jnp.where(col < valid, x, 0.0)
    xs = jnp.concatenate([x[:, q * QTR:(q + 1) * QTR] for q in range(4)],
                         axis=0)
    eye = jnp.float32(
        lax.broadcasted_iota(jnp.int32, (SUPER, SUPER), 0)
        == lax.broadcasted_iota(jnp.int32, (SUPER, SUPER), 1))
    hi = xs.astype(jnp.bfloat16)
    lo = (xs - hi.astype(jnp.float32)).astype(jnp.bfloat16)
    dims = (((0,), (0,)), ((), ()))
    out_ref[...] = (
        lax.dot_general(hi, eye.astype(jnp.bfloat16), dims,
                        preferred_element_type=jnp.float32)
        + lax.dot_general(lo, eye.astype(jnp.bfloat16), dims,
                          preferred_element_type=jnp.float32))


def _tc_relayout(tabT):
    rows = tabT.shape[1]
    n_blocks = pl.cdiv(rows, BLK)
    return pl.pallas_call(
        functools.partial(_relayout_body, rows),
        grid=(n_blocks,),
        in_specs=[pl.BlockSpec((EMBED_DIM, BLK), lambda i: (0, i))],
        out_specs=pl.BlockSpec((QTR, SUPER), lambda i: (i, 0)),
        out_shape=jax.ShapeDtypeStruct((n_blocks * QTR, SUPER), jnp.float32),
    )(tabT)


def _sc_gather(item_id3, org_id3, item_tab4, org_tab4, n_workers, n_chunks):
    b_per_w = n_chunks * CHUNK
    mesh = plsc.VectorSubcoreMesh(core_axis_name="c", subcore_axis_name="s")

    @functools.partial(
        pl.kernel,
        out_type=(
            jax.ShapeDtypeStruct((BATCH, SUPER), jnp.float32),
            jax.ShapeDtypeStruct((BATCH, SUPER), jnp.float32),
        ),
        mesh=mesh,
        scratch_types=[
            pltpu.VMEM((n_chunks, CHUNK), jnp.int32),
            pltpu.VMEM((n_chunks, CHUNK), jnp.int32),
            pltpu.VMEM((2, CHUNK, SUPER), jnp.float32),
            pltpu.VMEM((2, CHUNK, SUPER), jnp.float32),
            pltpu.SemaphoreType.DMA,
            pltpu.SemaphoreType.DMA,
        ],
    )
    def k(iid_hbm, oid_hbm, itab_hbm, otab_hbm, iout_hbm, oout_hbm,
          iidx_v, oidx_v, ibuf_v, obuf_v, gsem, osem):
        wid = lax.axis_index("s") * 2 + lax.axis_index("c")
        base = wid * b_per_w
        pltpu.sync_copy(iid_hbm.at[wid], iidx_v)
        pltpu.sync_copy(oid_hbm.at[wid], oidx_v)
        outs = []
        for j in range(n_chunks):
            s = j % 2
            if j >= 2:
                outs[2 * (j - 2)].wait()
                outs[2 * (j - 2) + 1].wait()
            g1 = pltpu.async_copy(
                itab_hbm.at[iidx_v.at[j]], ibuf_v.at[s], gsem)
            g2 = pltpu.async_copy(
                otab_hbm.at[oidx_v.at[j]], obuf_v.at[s], gsem)
            g1.wait()
            g2.wait()
            dst = pl.ds(base + j * CHUNK, CHUNK)
            outs.append(pltpu.async_copy(
                ibuf_v.at[s], iout_hbm.at[dst], osem))
            outs.append(pltpu.async_copy(
                obuf_v.at[s], oout_hbm.at[dst], osem))
        for c in outs[-4:]:
            c.wait()

    return k(item_id3, org_id3, item_tab4, org_tab4)


def _mlp_body(ig_ref, og_ref, isel_ref, osel_ref,
              w1i_ref, w1o_ref, b1_ref, w2_ref, b2_ref, w3_ref, b3_ref,
              out_ref):
    ig = ig_ref[...]
    grp = lax.broadcasted_iota(jnp.int32, ig.shape, 1) >> 5
    ig = jnp.where(grp == isel_ref[...], ig, 0.0)
    og = jnp.where(grp == osel_ref[...], og_ref[...], 0.0)
    x = jnp.maximum(
        jnp.dot(ig, w1i_ref[...], preferred_element_type=jnp.float32)
        + jnp.dot(og, w1o_ref[...], preferred_element_type=jnp.float32)
        + b1_ref[...], 0.0)
    x = jnp.maximum(
        jnp.dot(x, w2_ref[...], preferred_element_type=jnp.float32)
        + b2_ref[...], 0.0)
    y = jnp.dot(x, w3_ref[...], preferred_element_type=jnp.float32) + b3_ref[...]
    out_ref[...] = jax.nn.sigmoid(y)


def _tc_mlp(ig, og, isel, osel, W1, b1, W2, b2, W3, b3, block_b=2048):
    n_blocks = BATCH // block_b
    full = lambda shape: pl.BlockSpec(shape, lambda i: (0, 0))
    return pl.pallas_call(
        _mlp_body,
        grid=(n_blocks,),
        in_specs=[
            pl.BlockSpec((block_b, SUPER), lambda i: (i, 0)),
            pl.BlockSpec((block_b, SUPER), lambda i: (i, 0)),
            pl.BlockSpec((block_b, 1), lambda i: (i, 0)),
            pl.BlockSpec((block_b, 1), lambda i: (i, 0)),
            full((SUPER, 128)),
            full((SUPER, 128)),
            full((1, 128)),
            full((128, 64)),
            full((1, 64)),
            full((64, 1)),
            full((1, 1)),
        ],
        out_specs=pl.BlockSpec((block_b, 1), lambda i: (i, 0)),
        out_shape=jax.ShapeDtypeStruct((BATCH, 1), jnp.float32),
    )(ig, og, isel, osel, jnp.tile(W1[:EMBED_DIM], (4, 1)),
      jnp.tile(W1[EMBED_DIM:], (4, 1)), b1.reshape(1, -1), W2,
      b2.reshape(1, -1), W3, b3.reshape(1, -1))


def kernel(item_id, org_id, item_table, org_table, W1, b1, W2, b2, W3, b3):
    info = plsc.get_sparse_core_info()
    n_workers = info.num_cores * info.num_subcores
    n_chunks = BATCH // (n_workers * CHUNK)
    item_id = item_id.astype(jnp.int32)
    org_id = org_id.astype(jnp.int32)
    isup = ((item_id // BLK) * QTR) + (item_id % QTR)
    osup = ((org_id // BLK) * QTR) + (org_id % QTR)
    item_id3 = isup.reshape(n_workers, n_chunks, CHUNK)
    org_id3 = osup.reshape(n_workers, n_chunks, CHUNK)
    item_tab4 = _tc_relayout(item_table.T)
    org_tab4 = _tc_relayout(org_table.T)
    ig, og = _sc_gather(item_id3, org_id3, item_tab4, org_tab4,
                        n_workers, n_chunks)
    isel = ((item_id % BLK) // QTR).reshape(BATCH, 1)
    osel = ((org_id % BLK) // QTR).reshape(BATCH, 1)
    return _tc_mlp(ig, og, isel, osel, W1, b1, W2, b2, W3, b3)


# speedup vs baseline: 4.6276x; 1.0273x over previous
import functools

import jax
import jax.numpy as jnp
from jax import lax
from jax.experimental import pallas as pl
from jax.experimental.pallas import tpu as pltpu
from jax.experimental.pallas import tpu_sc as plsc

BATCH = 16384
EMBED_DIM = 32
SUPER = 128
CHUNK = 128


BLK = 16384
QTR = BLK // 4


def _relayout_body(rows, tT_ref, out_ref):
    x = tT_ref[...]
    valid = rows - pl.program_id(0) * BLK
    col = lax.broadcasted_iota(jnp.int32, x.shape, 1)
    x = jnp.where(col < valid, x, 0.0)
    xs = jnp.concatenate([x[:, q * QTR:(q + 1) * QTR] for q in range(4)],
                         axis=0)
    eye = jnp.float32(
        lax.broadcasted_iota(jnp.int32, (SUPER, SUPER), 0)
        == lax.broadcasted_iota(jnp.int32, (SUPER, SUPER), 1))
    hi = xs.astype(jnp.bfloat16)
    lo = (xs - hi.astype(jnp.float32)).astype(jnp.bfloat16)
    dims = (((0,), (0,)), ((), ()))
    out_ref[...] = (
        lax.dot_general(hi, eye.astype(jnp.bfloat16), dims,
                        preferred_element_type=jnp.float32)
        + lax.dot_general(lo, eye.astype(jnp.bfloat16), dims,
                          preferred_element_type=jnp.float32))


def _tc_relayout(tabT):
    rows = tabT.shape[1]
    n_blocks = pl.cdiv(rows, BLK)
    return pl.pallas_call(
        functools.partial(_relayout_body, rows),
        grid=(n_blocks,),
        in_specs=[pl.BlockSpec((EMBED_DIM, BLK), lambda i: (0, i))],
        out_specs=pl.BlockSpec((QTR, SUPER), lambda i: (i, 0)),
        out_shape=jax.ShapeDtypeStruct((n_blocks * QTR, SUPER), jnp.float32),
    )(tabT)


def _sc_gather(item_id3, org_id3, item_tab4, org_tab4, n_workers, n_chunks):
    b_per_w = n_chunks * CHUNK
    mesh = plsc.VectorSubcoreMesh(core_axis_name="c", subcore_axis_name="s")

    @functools.partial(
        pl.kernel,
        out_type=(
            jax.ShapeDtypeStruct((BATCH, SUPER), jnp.float32),
            jax.ShapeDtypeStruct((BATCH, SUPER), jnp.float32),
        ),
        mesh=mesh,
        scratch_types=[
            pltpu.VMEM((n_chunks, CHUNK), jnp.int32),
            pltpu.VMEM((n_chunks, CHUNK), jnp.int32),
            pltpu.VMEM((2, CHUNK, SUPER), jnp.float32),
            pltpu.VMEM((2, CHUNK, SUPER), jnp.float32),
            pltpu.SemaphoreType.DMA,
            pltpu.SemaphoreType.DMA,
        ],
    )
    def k(iid_hbm, oid_hbm, itab_hbm, otab_hbm, iout_hbm, oout_hbm,
          iidx_v, oidx_v, ibuf_v, obuf_v, gsem, osem):
        wid = lax.axis_index("s") * 2 + lax.axis_index("c")
        base = wid * b_per_w
        pltpu.sync_copy(iid_hbm.at[wid], iidx_v)
        pltpu.sync_copy(oid_hbm.at[wid], oidx_v)
        outs = []
        for j in range(n_chunks):
            s = j % 2
            if j >= 2:
                outs[2 * (j - 2)].wait()
                outs[2 * (j - 2) + 1].wait()
            g1 = pltpu.async_copy(
                itab_hbm.at[iidx_v.at[j]], ibuf_v.at[s], gsem)
            g2 = pltpu.async_copy(
                otab_hbm.at[oidx_v.at[j]], obuf_v.at[s], gsem)
            g1.wait()
            g2.wait()
            dst = pl.ds(base + j * CHUNK, CHUNK)
            outs.append(pltpu.async_copy(
                ibuf_v.at[s], iout_hbm.at[dst], osem))
            outs.append(pltpu.async_copy(
                obuf_v.at[s], oout_hbm.at[dst], osem))
        for c in outs[-4:]:
            c.wait()

    return k(item_id3, org_id3, item_tab4, org_tab4)


def _mlp_body(ig_ref, og_ref, isel_ref, osel_ref,
              w1i_ref, w1o_ref, b1_ref, w2_ref, b2_ref, w3_ref, b3_ref,
              out_ref):
    ig = ig_ref[...]
    grp = lax.broadcasted_iota(jnp.int32, ig.shape, 1) >> 5
    ig = jnp.where(grp == jnp.int32(isel_ref[...]), ig, 0.0)
    og = jnp.where(grp == jnp.int32(osel_ref[...]), og_ref[...], 0.0)
    x = jnp.maximum(
        jnp.dot(ig, w1i_ref[...], preferred_element_type=jnp.float32)
        + jnp.dot(og, w1o_ref[...], preferred_element_type=jnp.float32)
        + b1_ref[...], 0.0)
    x = jnp.maximum(
        jnp.dot(x, w2_ref[...], preferred_element_type=jnp.float32)
        + b2_ref[...], 0.0)
    y = jnp.dot(x, w3_ref[...], preferred_element_type=jnp.float32) + b3_ref[...]
    out_ref[...] = jax.nn.sigmoid(y)


def _tc_mlp(ig, og, isel, osel, W1, b1, W2, b2, W3, b3, block_b=2048):
    n_blocks = BATCH // block_b
    full = lambda shape: pl.BlockSpec(shape, lambda i: (0, 0))
    return pl.pallas_call(
        _mlp_body,
        grid=(n_blocks,),
        in_specs=[
            pl.BlockSpec((block_b, SUPER), lambda i: (i, 0)),
            pl.BlockSpec((block_b, SUPER), lambda i: (i, 0)),
            pl.BlockSpec((block_b, 1), lambda i: (i, 0)),
            pl.BlockSpec((block_b, 1), lambda i: (i, 0)),
            full((SUPER, 128)),
            full((SUPER, 128)),
            full((1, 128)),
            full((128, 64)),
            full((1, 64)),
            full((64, 1)),
            full((1, 1)),
        ],
        out_specs=pl.BlockSpec((block_b, 1), lambda i: (i, 0)),
        out_shape=jax.ShapeDtypeStruct((BATCH, 1), jnp.float32),
    )(ig, og, isel, osel, jnp.tile(W1[:EMBED_DIM], (4, 1)),
      jnp.tile(W1[EMBED_DIM:], (4, 1)), b1.reshape(1, -1), W2,
      b2.reshape(1, -1), W3, b3.reshape(1, -1))


def kernel(item_id, org_id, item_table, org_table, W1, b1, W2, b2, W3, b3):
    info = plsc.get_sparse_core_info()
    n_workers = info.num_cores * info.num_subcores
    n_chunks = BATCH // (n_workers * CHUNK)
    item_id = item_id.astype(jnp.int32)
    org_id = org_id.astype(jnp.int32)
    isup = ((item_id // BLK) * QTR) + (item_id % QTR)
    osup = ((org_id // BLK) * QTR) + (org_id % QTR)
    item_id3 = isup.reshape(n_workers, n_chunks, CHUNK)
    org_id3 = osup.reshape(n_workers, n_chunks, CHUNK)
    item_tab4 = _tc_relayout(item_table.T)
    org_tab4 = _tc_relayout(org_table.T)
    ig, og = _sc_gather(item_id3, org_id3, item_tab4, org_tab4,
                        n_workers, n_chunks)
    isel = ((item_id % BLK) // QTR).astype(jnp.int8).reshape(BATCH, 1)
    osel = ((org_id % BLK) // QTR).astype(jnp.int8).reshape(BATCH, 1)
    return _tc_mlp(ig, og, isel, osel, W1, b1, W2, b2, W3, b3)


# speedup vs baseline: 5.1894x; 1.1214x over previous
import functools

import jax
import jax.numpy as jnp
from jax import lax
from jax.experimental import pallas as pl
from jax.experimental.pallas import tpu as pltpu
from jax.experimental.pallas import tpu_sc as plsc

BATCH = 16384
EMBED_DIM = 32
SUPER = 128
CHUNK = 128


BLK = 32768
QTR = BLK // 4


def _relayout_body(rows, tT_ref, out_ref):
    x = tT_ref[...]
    valid = rows - pl.program_id(0) * BLK
    col = lax.broadcasted_iota(jnp.int32, x.shape, 1)
    x = jnp.where(col < valid, x, 0.0)
    xs = jnp.concatenate([x[:, q * QTR:(q + 1) * QTR] for q in range(4)],
                         axis=0)
    eye = jnp.float32(
        lax.broadcasted_iota(jnp.int32, (SUPER, SUPER), 0)
        == lax.broadcasted_iota(jnp.int32, (SUPER, SUPER), 1))
    hi = xs.astype(jnp.bfloat16)
    lo = (xs - hi.astype(jnp.float32)).astype(jnp.bfloat16)
    dims = (((0,), (0,)), ((), ()))
    out_ref[...] = (
        lax.dot_general(hi, eye.astype(jnp.bfloat16), dims,
                        preferred_element_type=jnp.float32)
        + lax.dot_general(lo, eye.astype(jnp.bfloat16), dims,
                          preferred_element_type=jnp.float32))


def _tc_relayout(tabT):
    rows = tabT.shape[1]
    n_blocks = pl.cdiv(rows, BLK)
    return pl.pallas_call(
        functools.partial(_relayout_body, rows),
        grid=(n_blocks,),
        in_specs=[pl.BlockSpec((EMBED_DIM, BLK), lambda i: (0, i))],
        out_specs=pl.BlockSpec((QTR, SUPER), lambda i: (i, 0)),
        out_shape=jax.ShapeDtypeStruct((n_blocks * QTR, SUPER), jnp.float32),
    )(tabT)


def _sc_gather(item_id3, org_id3, item_tab4, org_tab4, n_workers, n_chunks):
    b_per_w = n_chunks * CHUNK
    mesh = plsc.VectorSubcoreMesh(core_axis_name="c", subcore_axis_name="s")

    @functools.partial(
        pl.kernel,
        out_type=(
            jax.ShapeDtypeStruct((BATCH, SUPER), jnp.float32),
            jax.ShapeDtypeStruct((BATCH, SUPER), jnp.float32),
        ),
        mesh=mesh,
        scratch_types=[
            pltpu.VMEM((n_chunks, CHUNK), jnp.int32),
            pltpu.VMEM((n_chunks, CHUNK), jnp.int32),
            pltpu.VMEM((2, CHUNK, SUPER), jnp.float32),
            pltpu.VMEM((2, CHUNK, SUPER), jnp.float32),
            pltpu.SemaphoreType.DMA,
            pltpu.SemaphoreType.DMA,
        ],
    )
    def k(iid_hbm, oid_hbm, itab_hbm, otab_hbm, iout_hbm, oout_hbm,
          iidx_v, oidx_v, ibuf_v, obuf_v, gsem, osem):
        wid = lax.axis_index("s") * 2 + lax.axis_index("c")
        base = wid * b_per_w
        pltpu.sync_copy(iid_hbm.at[wid], iidx_v)
        pltpu.sync_copy(oid_hbm.at[wid], oidx_v)
        outs = []
        for j in range(n_chunks):
            s = j % 2
            if j >= 2:
                outs[2 * (j - 2)].wait()
                outs[2 * (j - 2) + 1].wait()
            g1 = pltpu.async_copy(
                itab_hbm.at[iidx_v.at[j]], ibuf_v.at[s], gsem)
            g2 = pltpu.async_copy(
                otab_hbm.at[oidx_v.at[j]], obuf_v.at[s], gsem)
            g1.wait()
            g2.wait()
            dst = pl.ds(base + j * CHUNK, CHUNK)
            outs.append(pltpu.async_copy(
                ibuf_v.at[s], iout_hbm.at[dst], osem))
            outs.append(pltpu.async_copy(
                obuf_v.at[s], oout_hbm.at[dst], osem))
        for c in outs[-4:]:
            c.wait()

    return k(item_id3, org_id3, item_tab4, org_tab4)


def _mlp_body(ig_ref, og_ref, isel_ref, osel_ref,
              w1i_ref, w1o_ref, b1_ref, w2_ref, b2_ref, w3_ref, b3_ref,
              out_ref):
    ig = ig_ref[...]
    grp = lax.broadcasted_iota(jnp.int32, ig.shape, 1) >> 5
    ig = jnp.where(grp == jnp.int32(isel_ref[...]), ig, 0.0)
    og = jnp.where(grp == jnp.int32(osel_ref[...]), og_ref[...], 0.0)
    x = jnp.maximum(
        jnp.dot(ig, w1i_ref[...], preferred_element_type=jnp.float32)
        + jnp.dot(og, w1o_ref[...], preferred_element_type=jnp.float32)
        + b1_ref[...], 0.0)
    x = jnp.maximum(
        jnp.dot(x, w2_ref[...], preferred_element_type=jnp.float32)
        + b2_ref[...], 0.0)
    y = jnp.dot(x, w3_ref[...], preferred_element_type=jnp.float32) + b3_ref[...]
    out_ref[...] = jax.nn.sigmoid(y)


def _tc_mlp(ig, og, isel, osel, W1, b1, W2, b2, W3, b3, block_b=2048):
    n_blocks = BATCH // block_b
    full = lambda shape: pl.BlockSpec(shape, lambda i: (0, 0))
    return pl.pallas_call(
        _mlp_body,
        grid=(n_blocks,),
        in_specs=[
            pl.BlockSpec((block_b, SUPER), lambda i: (i, 0)),
            pl.BlockSpec((block_b, SUPER), lambda i: (i, 0)),
            pl.BlockSpec((block_b, 1), lambda i: (i, 0)),
            pl.BlockSpec((block_b, 1), lambda i: (i, 0)),
            full((SUPER, 128)),
            full((SUPER, 128)),
            full((1, 128)),
            full((128, 64)),
            full((1, 64)),
            full((64, 1)),
            full((1, 1)),
        ],
        out_specs=pl.BlockSpec((block_b, 1), lambda i: (i, 0)),
        out_shape=jax.ShapeDtypeStruct((BATCH, 1), jnp.float32),
    )(ig, og, isel, osel, jnp.tile(W1[:EMBED_DIM], (4, 1)),
      jnp.tile(W1[EMBED_DIM:], (4, 1)), b1.reshape(1, -1), W2,
      b2.reshape(1, -1), W3, b3.reshape(1, -1))


def kernel(item_id, org_id, item_table, org_table, W1, b1, W2, b2, W3, b3):
    info = plsc.get_sparse_core_info()
    n_workers = info.num_cores * info.num_subcores
    n_chunks = BATCH // (n_workers * CHUNK)
    item_id = item_id.astype(jnp.int32)
    org_id = org_id.astype(jnp.int32)
    isup = ((item_id // BLK) * QTR) + (item_id % QTR)
    osup = ((org_id // BLK) * QTR) + (org_id % QTR)
    item_id3 = isup.reshape(n_workers, n_chunks, CHUNK)
    org_id3 = osup.reshape(n_workers, n_chunks, CHUNK)
    item_tab4 = _tc_relayout(item_table.T)
    org_tab4 = _tc_relayout(org_table.T)
    ig, og = _sc_gather(item_id3, org_id3, item_tab4, org_tab4,
                        n_workers, n_chunks)
    isel = ((item_id % BLK) // QTR).astype(jnp.int8).reshape(BATCH, 1)
    osel = ((org_id % BLK) // QTR).astype(jnp.int8).reshape(BATCH, 1)
    return _tc_mlp(ig, og, isel, osel, W1, b1, W2, b2, W3, b3)


# speedup vs baseline: 5.2898x; 1.0193x over previous
import functools

import jax
import jax.numpy as jnp
from jax import lax
from jax.experimental import pallas as pl
from jax.experimental.pallas import tpu as pltpu
from jax.experimental.pallas import tpu_sc as plsc

BATCH = 16384
EMBED_DIM = 32
SUPER = 128
CHUNK = 128


BLK = 65536
QTR = BLK // 4


def _relayout_body(rows, tT_ref, out_ref):
    x = tT_ref[...]
    valid = rows - pl.program_id(0) * BLK
    col = lax.broadcasted_iota(jnp.int32, x.shape, 1)
    x = jnp.where(col < valid, x, 0.0)
    xs = jnp.concatenate([x[:, q * QTR:(q + 1) * QTR] for q in range(4)],
                         axis=0)
    eye = jnp.float32(
        lax.broadcasted_iota(jnp.int32, (SUPER, SUPER), 0)
        == lax.broadcasted_iota(jnp.int32, (SUPER, SUPER), 1))
    hi = xs.astype(jnp.bfloat16)
    lo = (xs - hi.astype(jnp.float32)).astype(jnp.bfloat16)
    dims = (((0,), (0,)), ((), ()))
    out_ref[...] = (
        lax.dot_general(hi, eye.astype(jnp.bfloat16), dims,
                        preferred_element_type=jnp.float32)
        + lax.dot_general(lo, eye.astype(jnp.bfloat16), dims,
                          preferred_element_type=jnp.float32))


def _tc_relayout(tabT):
    rows = tabT.shape[1]
    n_blocks = pl.cdiv(rows, BLK)
    return pl.pallas_call(
        functools.partial(_relayout_body, rows),
        grid=(n_blocks,),
        in_specs=[pl.BlockSpec((EMBED_DIM, BLK), lambda i: (0, i))],
        out_specs=pl.BlockSpec((QTR, SUPER), lambda i: (i, 0)),
        out_shape=jax.ShapeDtypeStruct((n_blocks * QTR, SUPER), jnp.float32),
    )(tabT)


def _sc_gather(item_id3, org_id3, item_tab4, org_tab4, n_workers, n_chunks):
    b_per_w = n_chunks * CHUNK
    mesh = plsc.VectorSubcoreMesh(core_axis_name="c", subcore_axis_name="s")

    @functools.partial(
        pl.kernel,
        out_type=(
            jax.ShapeDtypeStruct((BATCH, SUPER), jnp.float32),
            jax.ShapeDtypeStruct((BATCH, SUPER), jnp.float32),
        ),
        mesh=mesh,
        scratch_types=[
            pltpu.VMEM((n_chunks, CHUNK), jnp.int32),
            pltpu.VMEM((n_chunks, CHUNK), jnp.int32),
            pltpu.VMEM((2, CHUNK, SUPER), jnp.float32),
            pltpu.VMEM((2, CHUNK, SUPER), jnp.float32),
            pltpu.SemaphoreType.DMA,
            pltpu.SemaphoreType.DMA,
        ],
    )
    def k(iid_hbm, oid_hbm, itab_hbm, otab_hbm, iout_hbm, oout_hbm,
          iidx_v, oidx_v, ibuf_v, obuf_v, gsem, osem):
        wid = lax.axis_index("s") * 2 + lax.axis_index("c")
        base = wid * b_per_w
        pltpu.sync_copy(iid_hbm.at[wid], iidx_v)
        pltpu.sync_copy(oid_hbm.at[wid], oidx_v)
        outs = []
        for j in range(n_chunks):
            s = j % 2
            if j >= 2:
                outs[2 * (j - 2)].wait()
                outs[2 * (j - 2) + 1].wait()
            g1 = pltpu.async_copy(
                itab_hbm.at[iidx_v.at[j]], ibuf_v.at[s], gsem)
            g2 = pltpu.async_copy(
                otab_hbm.at[oidx_v.at[j]], obuf_v.at[s], gsem)
            g1.wait()
            g2.wait()
            dst = pl.ds(base + j * CHUNK, CHUNK)
            outs.append(pltpu.async_copy(
                ibuf_v.at[s], iout_hbm.at[dst], osem))
            outs.append(pltpu.async_copy(
                obuf_v.at[s], oout_hbm.at[dst], osem))
        for c in outs[-4:]:
            c.wait()

    return k(item_id3, org_id3, item_tab4, org_tab4)


def _mlp_body(ig_ref, og_ref, isel_ref, osel_ref,
              w1i_ref, w1o_ref, b1_ref, w2_ref, b2_ref, w3_ref, b3_ref,
              out_ref):
    ig = ig_ref[...]
    grp = lax.broadcasted_iota(jnp.int32, ig.shape, 1) >> 5
    ig = jnp.where(grp == jnp.int32(isel_ref[...]), ig, 0.0)
    og = jnp.where(grp == jnp.int32(osel_ref[...]), og_ref[...], 0.0)
    x = jnp.maximum(
        jnp.dot(ig, w1i_ref[...], preferred_element_type=jnp.float32)
        + jnp.dot(og, w1o_ref[...], preferred_element_type=jnp.float32)
        + b1_ref[...], 0.0)
    x = jnp.maximum(
        jnp.dot(x, w2_ref[...], preferred_element_type=jnp.float32)
        + b2_ref[...], 0.0)
    y = jnp.dot(x, w3_ref[...], preferred_element_type=jnp.float32) + b3_ref[...]
    out_ref[...] = jax.nn.sigmoid(y)


def _tc_mlp(ig, og, isel, osel, W1, b1, W2, b2, W3, b3, block_b=2048):
    n_blocks = BATCH // block_b
    full = lambda shape: pl.BlockSpec(shape, lambda i: (0, 0))
    return pl.pallas_call(
        _mlp_body,
        grid=(n_blocks,),
        in_specs=[
            pl.BlockSpec((block_b, SUPER), lambda i: (i, 0)),
            pl.BlockSpec((block_b, SUPER), lambda i: (i, 0)),
            pl.BlockSpec((block_b, 1), lambda i: (i, 0)),
            pl.BlockSpec((block_b, 1), lambda i: (i, 0)),
            full((SUPER, 128)),
            full((SUPER, 128)),
            full((1, 128)),
            full((128, 64)),
            full((1, 64)),
            full((64, 1)),
            full((1, 1)),
        ],
        out_specs=pl.BlockSpec((block_b, 1), lambda i: (i, 0)),
        out_shape=jax.ShapeDtypeStruct((BATCH, 1), jnp.float32),
    )(ig, og, isel, osel, jnp.tile(W1[:EMBED_DIM], (4, 1)),
      jnp.tile(W1[EMBED_DIM:], (4, 1)), b1.reshape(1, -1), W2,
      b2.reshape(1, -1), W3, b3.reshape(1, -1))


def kernel(item_id, org_id, item_table, org_table, W1, b1, W2, b2, W3, b3):
    info = plsc.get_sparse_core_info()
    n_workers = info.num_cores * info.num_subcores
    n_chunks = BATCH // (n_workers * CHUNK)
    item_id = item_id.astype(jnp.int32)
    org_id = org_id.astype(jnp.int32)
    isup = ((item_id // BLK) * QTR) + (item_id % QTR)
    osup = ((org_id // BLK) * QTR) + (org_id % QTR)
    item_id3 = isup.reshape(n_workers, n_chunks, CHUNK)
    org_id3 = osup.reshape(n_workers, n_chunks, CHUNK)
    item_tab4 = _tc_relayout(item_table.T)
    org_tab4 = _tc_relayout(org_table.T)
    ig, og = _sc_gather(item_id3, org_id3, item_tab4, org_tab4,
                        n_workers, n_chunks)
    isel = ((item_id % BLK) // QTR).astype(jnp.int8).reshape(BATCH, 1)
    osel = ((org_id % BLK) // QTR).astype(jnp.int8).reshape(BATCH, 1)
    return _tc_mlp(ig, og, isel, osel, W1, b1, W2, b2, W3, b3)
